# baseline (speedup 1.0000x reference)
import functools
import jax
import jax.numpy as jnp
from jax import lax
from jax.experimental import pallas as pl
from jax.experimental.pallas import tpu as pltpu
from jax.experimental.pallas import tpu_sc as plsc

N, E, H = 10000, 320000, 128
NC, NS, NW = 2, 16, 32
C = 128
EW = 10240
NCHUNK = EW // C
C2 = 64
NCH2 = EW // C2
EPAD = NW * EW
NPAD = NS * 640
DEGW = NPAD

ROW_BLK = 2000
EDGE_BLK = 8000

_sc_mesh = plsc.VectorSubcoreMesh(core_axis_name="c", subcore_axis_name="s")


@functools.partial(
    pl.kernel,
    out_type=jax.ShapeDtypeStruct((NC, DEGW), jnp.float32),
    mesh=_sc_mesh,
    scratch_types=[
        pltpu.VMEM((NCHUNK, C), jnp.int32),
        pltpu.VMEM((NCHUNK, C), jnp.float32),
        pltpu.VMEM_SHARED((DEGW,), jnp.float32),
        pltpu.VMEM((640,), jnp.float32),
    ],
)
def _deg_kernel(dst_hbm, ew_hbm, out_hbm, dstv, ewv, sdeg, zbuf):
    c = lax.axis_index("c")
    s = lax.axis_index("s")
    w = s * NC + c
    pltpu.sync_copy(dst_hbm.at[w], dstv)
    pltpu.sync_copy(ew_hbm.at[w], ewv)

    zvec = jnp.zeros((16,), jnp.float32)

    def zstore(i, carry):
        zbuf[pl.ds(i * 16, 16)] = zvec
        return carry

    lax.fori_loop(0, 40, zstore, 0)
    pltpu.sync_copy(zbuf, sdeg.at[pl.ds(s * 640, 640)])
    plsc.subcore_barrier()

    def chunk(j, carry):
        pltpu.sync_copy(ewv.at[j], sdeg.at[dstv.at[j]], add=True)
        return carry

    lax.fori_loop(0, NCHUNK, chunk, 0)
    plsc.subcore_barrier()
    pltpu.sync_copy(sdeg.at[pl.ds(s * 640, 640)], out_hbm.at[c, pl.ds(s * 640, 640)])


@functools.partial(
    pl.kernel,
    out_type=jax.ShapeDtypeStruct((NC, NPAD, H), jnp.float32),
    mesh=_sc_mesh,
    scratch_types=[
        pltpu.VMEM((EW // 256, 128), jnp.int32),
        pltpu.VMEM((EW // 256, 128), jnp.int32),
        pltpu.VMEM((NCHUNK, C), jnp.float32),
        pltpu.VMEM((C2, H), jnp.float32),
        pltpu.VMEM((C2, H), jnp.float32),
        pltpu.VMEM((C2, H), jnp.float32),
        pltpu.VMEM((C2,), jnp.int32),
        pltpu.VMEM((C2,), jnp.int32),
        pltpu.VMEM((C2,), jnp.int32),
        pltpu.VMEM((32,), jnp.int32),
        pltpu.VMEM((32,), jnp.int32),
        pltpu.VMEM((32,), jnp.int32),
        pltpu.VMEM((32,), jnp.int32),
        pltpu.VMEM((32,), jnp.int32),
        pltpu.VMEM((32,), jnp.int32),
        pltpu.VMEM_SHARED((NPAD, H), jnp.float32),
        pltpu.SemaphoreType.DMA,
        pltpu.SemaphoreType.DMA,
        pltpu.SemaphoreType.DMA,
        pltpu.SemaphoreType.DMA,
        pltpu.SemaphoreType.DMA,
        pltpu.SemaphoreType.DMA,
    ],
)
def _spmm_kernel(xs_hbm, srcp_hbm, dstp_hbm, ewr_hbm, out_hbm,
                 srcp, dstp, ewv, r0, r1, r2, st0, st1, st2,
                 d0a, d0b, d1a, d1b, d2a, d2b, sacc,
                 sg0, sg1, sg2, ss0, ss1, ss2):
    c = lax.axis_index("c")
    s = lax.axis_index("s")
    w = s * NC + c
    pltpu.async_copy(srcp_hbm.at[w], srcp, sg0)
    pltpu.async_copy(dstp_hbm.at[w], dstp, sg0)
    pltpu.async_copy(ewr_hbm.at[w], ewv, sg0)

    rowss = (r0, r1, r2)
    stages = (st0, st1, st2)
    dhalves = ((d0a, d0b), (d1a, d1b), (d2a, d2b))
    sgs = (sg0, sg1, sg2)
    sss = (ss0, ss1, ss2)
    zvec = jnp.zeros((16,), jnp.float32)
    m16 = jnp.full((16,), 0xFFFF, jnp.int32)
    sh16 = jnp.full((16,), 16, jnp.int32)

    def zrow(i, carry):
        for k in range(8):
            r0[i, pl.ds(k * 16, 16)] = zvec
        return carry

    lax.fori_loop(0, C2, zrow, 0)
    for t in range(10):
        pltpu.async_copy(r0, sacc.at[pl.ds(s * 640 + t * C2, C2)], sg1)
    for t in range(10):
        pltpu.make_async_copy(r0, sacc.at[pl.ds(s * 640, C2)], sg1).wait()
    pltpu.make_async_copy(srcp_hbm.at[0], srcp, sg0).wait()
    pltpu.make_async_copy(dstp_hbm.at[0], dstp, sg0).wait()
    pltpu.make_async_copy(ewr_hbm.at[0], ewv, sg0).wait()
    plsc.subcore_barrier()

    gdn = lax.GatherDimensionNumbers(
        offset_dims=(), collapsed_slice_dims=(0,), start_index_map=(0,))

    def unpack_idx(pk, j, blk):
        word0 = j * 32 + blk * 16
        v = pk[word0 // 128, pl.ds(word0 % 128, 16)]
        lo = jnp.bitwise_and(v, m16)
        hi = lax.shift_right_logical(v, sh16)
        return lo, hi

    def start_gather(j, r):
        sb = stages[r]
        for blk in range(2):
            lo, hi = unpack_idx(srcp, j, blk)
            sb[pl.ds(blk * 32, 16)] = lo
            sb[pl.ds(blk * 32 + 16, 16)] = hi
        pltpu.async_copy(xs_hbm.at[sb], rowss[r], sgs[r])

    def wait_gather(r):
        pltpu.make_async_copy(xs_hbm.at[stages[r]], rowss[r], sgs[r]).wait()

    def wait_scatter(r):
        pltpu.make_async_copy(rowss[r], sacc.at[stages[r]], sss[r]).wait()

    def scale_half(j, r, blk):
        rb = rowss[r]
        for g in range(2):
            col = (j % 2) * C2 + blk * 32 + g * 16
            ew16 = ewv[j // 2, pl.ds(col, 16)]
            for l in range(16):
                lidx = jnp.full((16, 1), l, jnp.int32)
                ewvec = lax.gather(
                    ew16, lidx, gdn, (1,),
                    mode=lax.GatherScatterMode.PROMISE_IN_BOUNDS)
                e = blk * 32 + g * 16 + l
                for k in range(8):
                    sl = pl.ds(k * 16, 16)
                    rb[e, sl] = rb[e, sl] * ewvec
        db = dhalves[r][blk]
        lo, hi = unpack_idx(dstp, j, blk)
        db[pl.ds(0, 16)] = lo
        db[pl.ds(16, 16)] = hi
        pltpu.async_copy(rb.at[pl.ds(blk * 32, 32)],
                         sacc.at[db], sss[r], add=True)

    def body(i, r, first=False, last=False):
        r2_ = (r + 2) % 3
        wait_gather(r)
        scale_half(i, r, 0)
        scale_half(i, r, 1)
        if not last:
            if not first:
                wait_scatter(r2_)
            start_gather(i + 2, r2_)

    start_gather(0, 0)
    start_gather(1, 1)
    body(0, 0, first=True)
    body(1, 1)

    def steady(t, carry):
        i0 = 2 + 3 * t
        for b in range(3):
            body(i0 + b, (2 + b) % 3)
        return carry

    lax.fori_loop(0, (NCH2 - 4) // 3, steady, 0)

    body(NCH2 - 2, (NCH2 - 2) % 3, last=True)
    body(NCH2 - 1, (NCH2 - 1) % 3, last=True)
    for r in range(3):
        wait_scatter(r)

    plsc.subcore_barrier()
    for t in range(5):
        sl = pl.ds(s * 640 + t * 128, 128)
        pltpu.sync_copy(sacc.at[sl], out_hbm.at[c, sl])


def _k1a_body(x_ref, winT_ref, bin_ref, h0_ref):
    h = jnp.dot(x_ref[...], winT_ref[...], preferred_element_type=jnp.float32)
    h0_ref[...] = jnp.maximum(h + bin_ref[...], 0.0)


def _k1a(x, WinT, b_in):
    grid = N // ROW_BLK
    return pl.pallas_call(
        _k1a_body,
        grid=(grid,),
        in_specs=[
            pl.BlockSpec((ROW_BLK, H), lambda i: (i, 0)),
            pl.BlockSpec((H, H), lambda i: (0, 0)),
            pl.BlockSpec((1, H), lambda i: (0, 0)),
        ],
        out_specs=pl.BlockSpec((ROW_BLK, H), lambda i: (i, 0)),
        out_shape=jax.ShapeDtypeStruct((N, H), jnp.float32),
    )(x, WinT, b_in.reshape(1, H))


def _k1b_body(h_ref, wc0T_ref, d0_ref, d1_ref, xs_ref, dinv_ref):
    dinv = lax.rsqrt(1.0 + d0_ref[...] + d1_ref[...])
    dinv_ref[...] = dinv
    xs_ref[...] = dinv * jnp.dot(h_ref[...], wc0T_ref[...],
                                 preferred_element_type=jnp.float32)


def _k1b(h, Wc0T, d0, d1):
    grid = N // ROW_BLK
    return pl.pallas_call(
        _k1b_body,
        grid=(grid,),
        in_specs=[
            pl.BlockSpec((ROW_BLK, H), lambda i: (i, 0)),
            pl.BlockSpec((H, H), lambda i: (0, 0)),
            pl.BlockSpec((ROW_BLK, 1), lambda i: (i, 0)),
            pl.BlockSpec((ROW_BLK, 1), lambda i: (i, 0)),
        ],
        out_specs=[
            pl.BlockSpec((ROW_BLK, H), lambda i: (i, 0)),
            pl.BlockSpec((ROW_BLK, 1), lambda i: (i, 0)),
        ],
        out_shape=[
            jax.ShapeDtypeStruct((N, H), jnp.float32),
            jax.ShapeDtypeStruct((N, 1), jnp.float32),
        ],
    )(h, Wc0T, d0, d1)


def _k3_body(s0_ref, s1_ref, xs_ref, h_ref, b_ref, wT_ref, dinv_ref,
             hn_ref, xsn_ref):
    dinv = dinv_ref[...]
    t = dinv * (s0_ref[...] + s1_ref[...] + xs_ref[...]) + b_ref[...] + h_ref[...]
    hn = jnp.maximum(t, 0.0)
    hn_ref[...] = hn
    xsn_ref[...] = dinv * jnp.dot(hn, wT_ref[...],
                                  preferred_element_type=jnp.float32)


def _k3(S0, S1, xs, h, b, WT, dinv):
    grid = N // ROW_BLK
    return pl.pallas_call(
        _k3_body,
        grid=(grid,),
        in_specs=[
            pl.BlockSpec((ROW_BLK, H), lambda i: (i, 0)),
            pl.BlockSpec((ROW_BLK, H), lambda i: (i, 0)),
            pl.BlockSpec((ROW_BLK, H), lambda i: (i, 0)),
            pl.BlockSpec((ROW_BLK, H), lambda i: (i, 0)),
            pl.BlockSpec((1, H), lambda i: (0, 0)),
            pl.BlockSpec((H, H), lambda i: (0, 0)),
            pl.BlockSpec((ROW_BLK, 1), lambda i: (i, 0)),
        ],
        out_specs=[
            pl.BlockSpec((ROW_BLK, H), lambda i: (i, 0)),
            pl.BlockSpec((ROW_BLK, H), lambda i: (i, 0)),
        ],
        out_shape=[
            jax.ShapeDtypeStruct((N, H), jnp.float32),
            jax.ShapeDtypeStruct((N, H), jnp.float32),
        ],
    )(S0, S1, xs, h, b.reshape(1, H), WT, dinv)


def _k4_body(s0_ref, s1_ref, xs_ref, h_ref, b_ref, dinv_ref, woT_ref,
             bo_ref, o_ref):
    dinv = dinv_ref[...]
    t = dinv * (s0_ref[...] + s1_ref[...] + xs_ref[...]) + b_ref[...] + h_ref[...]
    hn = jnp.maximum(t, 0.0)
    o_ref[...] = jnp.dot(hn, woT_ref[...],
                         preferred_element_type=jnp.float32) + bo_ref[...]


def _k4(S0, S1, xs, h, b, dinv, WoutT, bout):
    grid = N // ROW_BLK
    return pl.pallas_call(
        _k4_body,
        grid=(grid,),
        in_specs=[
            pl.BlockSpec((ROW_BLK, H), lambda i: (i, 0)),
            pl.BlockSpec((ROW_BLK, H), lambda i: (i, 0)),
            pl.BlockSpec((ROW_BLK, H), lambda i: (i, 0)),
            pl.BlockSpec((ROW_BLK, H), lambda i: (i, 0)),
            pl.BlockSpec((1, H), lambda i: (0, 0)),
            pl.BlockSpec((ROW_BLK, 1), lambda i: (i, 0)),
            pl.BlockSpec((H, H), lambda i: (0, 0)),
            pl.BlockSpec((1, H), lambda i: (0, 0)),
        ],
        out_specs=pl.BlockSpec((ROW_BLK, H), lambda i: (i, 0)),
        out_shape=jax.ShapeDtypeStruct((N, H), jnp.float32),
    )(S0, S1, xs, h, b.reshape(1, H), dinv, WoutT, bout.reshape(1, H))


def _edge_mlp_body(ea_ref, w1_ref, b1_ref, w2_ref, b2_ref, o_ref):
    a = jnp.dot(ea_ref[...], w1_ref[...], preferred_element_type=jnp.float32)
    a = jnp.maximum(a + b1_ref[...], 0.0)
    sc = jnp.dot(a, w2_ref[...], preferred_element_type=jnp.float32) + b2_ref[...]
    o_ref[...] = jax.nn.sigmoid(sc)


def _edge_mlp(edge_attr, We1, be1, We2, be2):
    grid = E // EDGE_BLK
    out = pl.pallas_call(
        _edge_mlp_body,
        grid=(grid,),
        in_specs=[
            pl.BlockSpec((EDGE_BLK, 16), lambda i: (i, 0)),
            pl.BlockSpec((16, 96), lambda i: (0, 0)),
            pl.BlockSpec((1, 96), lambda i: (0, 0)),
            pl.BlockSpec((96, 1), lambda i: (0, 0)),
            pl.BlockSpec((1, 1), lambda i: (0, 0)),
        ],
        out_specs=pl.BlockSpec((EDGE_BLK, 1), lambda i: (i, 0)),
        out_shape=jax.ShapeDtypeStruct((E, 1), jnp.float32),
    )(edge_attr, We1.T, be1.reshape(1, 96), We2.T, be2.reshape(1, 1))
    return out[:, 0]


def kernel(x, edge_index, edge_attr, Win, b_in, We1, be1, We2, be2,
           Wc0, bc0, Wc1, bc1, Wc2, bc2, Wout, bout):
    src = edge_index[0].astype(jnp.int32)
    dst = edge_index[1].astype(jnp.int32)

    ew = _edge_mlp(edge_attr, We1, be1, We2, be2)

    pad = EPAD - E
    zi = jnp.arange(pad, dtype=jnp.int32) % N
    srcr = jnp.concatenate([src, zi]).reshape(NW, NCHUNK, C)
    dstr = jnp.concatenate([dst, zi]).reshape(NW, NCHUNK, C)
    ewr = jnp.concatenate([ew, jnp.zeros((pad,), jnp.float32)]).reshape(
        NW, NCHUNK, C)

    h = _k1a(x, Win.T, b_in)

    degp = _deg_kernel(dstr, ewr)
    d0 = degp[0, :N].reshape(N, 1)
    d1 = degp[1, :N].reshape(N, 1)

    xs, dinv = _k1b(h, Wc0.T, d0, d1)

    def pack_u16(idx_flat):
        a = idx_flat.reshape(-1, 2, 16)
        words = a[:, 0, :] | (a[:, 1, :] << 16)
        return words.reshape(NW, EW // 256, 128)

    srcp = pack_u16(jnp.concatenate([src, zi]))
    dstp = pack_u16(jnp.concatenate([dst, zi]))

    for (W_next, b_cur) in ((Wc1, bc0), (Wc2, bc1)):
        S = _spmm_kernel(xs, srcp, dstp, ewr)
        h, xs = _k3(S[0, :N], S[1, :N], xs, h, b_cur, W_next.T, dinv)

    S = _spmm_kernel(xs, srcp, dstp, ewr)
    return _k4(S[0, :N], S[1, :N], xs, h, bc2, dinv, Wout.T, bout)


# speedup vs baseline: 1.0121x; 1.0121x over previous
import functools
import jax
import jax.numpy as jnp
from jax import lax
from jax.experimental import pallas as pl
from jax.experimental.pallas import tpu as pltpu
from jax.experimental.pallas import tpu_sc as plsc

N, E, H = 10000, 320000, 128
NC, NS, NW = 2, 16, 32
C = 128
EW = 10240
NCHUNK = EW // C
C2 = 64
NCH2 = EW // C2
EPAD = NW * EW
NPAD = NS * 640
DEGW = NPAD

ROW_BLK = 2000
EDGE_BLK = 8000

_sc_mesh = plsc.VectorSubcoreMesh(core_axis_name="c", subcore_axis_name="s")


@functools.partial(
    pl.kernel,
    out_type=jax.ShapeDtypeStruct((NC, DEGW), jnp.float32),
    mesh=_sc_mesh,
    scratch_types=[
        pltpu.VMEM((NCHUNK, C), jnp.int32),
        pltpu.VMEM((NCHUNK, C), jnp.float32),
        pltpu.VMEM_SHARED((DEGW,), jnp.float32),
        pltpu.VMEM((640,), jnp.float32),
    ],
)
def _deg_kernel(dst_hbm, ew_hbm, out_hbm, dstv, ewv, sdeg, zbuf):
    c = lax.axis_index("c")
    s = lax.axis_index("s")
    w = s * NC + c
    pltpu.sync_copy(dst_hbm.at[w], dstv)
    pltpu.sync_copy(ew_hbm.at[w], ewv)

    zvec = jnp.zeros((16,), jnp.float32)

    def zstore(i, carry):
        zbuf[pl.ds(i * 16, 16)] = zvec
        return carry

    lax.fori_loop(0, 40, zstore, 0)
    pltpu.sync_copy(zbuf, sdeg.at[pl.ds(s * 640, 640)])
    plsc.subcore_barrier()

    def chunk(j, carry):
        pltpu.sync_copy(ewv.at[j], sdeg.at[dstv.at[j]], add=True)
        return carry

    lax.fori_loop(0, NCHUNK, chunk, 0)
    plsc.subcore_barrier()
    pltpu.sync_copy(sdeg.at[pl.ds(s * 640, 640)], out_hbm.at[c, pl.ds(s * 640, 640)])


@functools.partial(
    pl.kernel,
    out_type=jax.ShapeDtypeStruct((NC, NPAD, H), jnp.float32),
    mesh=_sc_mesh,
    scratch_types=[
        pltpu.VMEM((EW // 256, 128), jnp.int32),
        pltpu.VMEM((EW // 256, 128), jnp.int32),
        pltpu.VMEM((NCHUNK, C), jnp.float32),
        pltpu.VMEM((C2, H), jnp.float32),
        pltpu.VMEM((C2, H), jnp.float32),
        pltpu.VMEM((C2, H), jnp.float32),
        pltpu.VMEM((C2,), jnp.int32),
        pltpu.VMEM((C2,), jnp.int32),
        pltpu.VMEM((C2,), jnp.int32),
        pltpu.VMEM((C2,), jnp.int32),
        pltpu.VMEM((C2,), jnp.int32),
        pltpu.VMEM((C2,), jnp.int32),
        pltpu.VMEM_SHARED((NPAD, H), jnp.float32),
        pltpu.SemaphoreType.DMA,
        pltpu.SemaphoreType.DMA,
        pltpu.SemaphoreType.DMA,
        pltpu.SemaphoreType.DMA,
        pltpu.SemaphoreType.DMA,
        pltpu.SemaphoreType.DMA,
    ],
)
def _spmm_kernel(xs_hbm, srcp_hbm, dstp_hbm, ewr_hbm, out_hbm,
                 srcp, dstp, ewv, r0, r1, r2, st0, st1, st2,
                 dt0, dt1, dt2, sacc,
                 sg0, sg1, sg2, ss0, ss1, ss2):
    c = lax.axis_index("c")
    s = lax.axis_index("s")
    w = s * NC + c
    pltpu.async_copy(srcp_hbm.at[w], srcp, sg0)
    pltpu.async_copy(dstp_hbm.at[w], dstp, sg0)
    pltpu.async_copy(ewr_hbm.at[w], ewv, sg0)

    rowss = (r0, r1, r2)
    stages = (st0, st1, st2)
    dstages = (dt0, dt1, dt2)
    sgs = (sg0, sg1, sg2)
    sss = (ss0, ss1, ss2)
    zvec = jnp.zeros((16,), jnp.float32)
    m16 = jnp.full((16,), 0xFFFF, jnp.int32)
    sh16 = jnp.full((16,), 16, jnp.int32)

    def zrow(i, carry):
        for k in range(8):
            r0[i, pl.ds(k * 16, 16)] = zvec
        return carry

    lax.fori_loop(0, C2, zrow, 0)
    for t in range(10):
        pltpu.async_copy(r0, sacc.at[pl.ds(s * 640 + t * C2, C2)], sg1)
    for t in range(10):
        pltpu.make_async_copy(r0, sacc.at[pl.ds(s * 640, C2)], sg1).wait()
    pltpu.make_async_copy(srcp_hbm.at[0], srcp, sg0).wait()
    pltpu.make_async_copy(dstp_hbm.at[0], dstp, sg0).wait()
    pltpu.make_async_copy(ewr_hbm.at[0], ewv, sg0).wait()
    plsc.subcore_barrier()

    gdn = lax.GatherDimensionNumbers(
        offset_dims=(), collapsed_slice_dims=(0,), start_index_map=(0,))

    def unpack_idx(pk, j, blk):
        word0 = j * 32 + blk * 16
        v = pk[word0 // 128, pl.ds(word0 % 128, 16)]
        lo = jnp.bitwise_and(v, m16)
        hi = lax.shift_right_logical(v, sh16)
        return lo, hi

    def start_gather(j, r):
        sb = stages[r]
        for blk in range(2):
            lo, hi = unpack_idx(srcp, j, blk)
            sb[pl.ds(blk * 32, 16)] = lo
            sb[pl.ds(blk * 32 + 16, 16)] = hi
        pltpu.async_copy(xs_hbm.at[sb], rowss[r], sgs[r])

    def wait_gather(r):
        pltpu.make_async_copy(xs_hbm.at[stages[r]], rowss[r], sgs[r]).wait()

    def wait_scatter(r):
        pltpu.make_async_copy(rowss[r], sacc.at[stages[r]], sss[r]).wait()

    def scale(j, r):
        rb = rowss[r]
        for g in range(C2 // 16):
            ew16 = ewv[j // 2, pl.ds((j % 2) * C2 + g * 16, 16)]
            for l in range(16):
                lidx = jnp.full((16, 1), l, jnp.int32)
                ewvec = lax.gather(
                    ew16, lidx, gdn, (1,),
                    mode=lax.GatherScatterMode.PROMISE_IN_BOUNDS)
                e = g * 16 + l
                for k in range(8):
                    sl = pl.ds(k * 16, 16)
                    rb[e, sl] = rb[e, sl] * ewvec

    def start_scatter(j, r):
        db = dstages[r]
        for blk in range(2):
            lo, hi = unpack_idx(dstp, j, blk)
            db[pl.ds(blk * 32, 16)] = lo
            db[pl.ds(blk * 32 + 16, 16)] = hi
        pltpu.async_copy(rowss[r], sacc.at[db], sss[r], add=True)

    def body(i, r, first=False, last=False):
        r2_ = (r + 2) % 3
        wait_gather(r)
        scale(i, r)
        start_scatter(i, r)
        if not last:
            if not first:
                wait_scatter(r2_)
            start_gather(i + 2, r2_)

    start_gather(0, 0)
    start_gather(1, 1)
    body(0, 0, first=True)
    body(1, 1)

    def steady(t, carry):
        i0 = 2 + 3 * t
        for b in range(3):
            body(i0 + b, (2 + b) % 3)
        return carry

    lax.fori_loop(0, (NCH2 - 4) // 3, steady, 0)

    body(NCH2 - 2, (NCH2 - 2) % 3, last=True)
    body(NCH2 - 1, (NCH2 - 1) % 3, last=True)
    for r in range(3):
        wait_scatter(r)

    plsc.subcore_barrier()
    for t in range(5):
        sl = pl.ds(s * 640 + t * 128, 128)
        pltpu.sync_copy(sacc.at[sl], out_hbm.at[c, sl])


def _k1a_body(x_ref, winT_ref, bin_ref, h0_ref):
    h = jnp.dot(x_ref[...], winT_ref[...], preferred_element_type=jnp.float32)
    h0_ref[...] = jnp.maximum(h + bin_ref[...], 0.0)


def _k1a(x, WinT, b_in):
    grid = N // ROW_BLK
    return pl.pallas_call(
        _k1a_body,
        grid=(grid,),
        in_specs=[
            pl.BlockSpec((ROW_BLK, H), lambda i: (i, 0)),
            pl.BlockSpec((H, H), lambda i: (0, 0)),
            pl.BlockSpec((1, H), lambda i: (0, 0)),
        ],
        out_specs=pl.BlockSpec((ROW_BLK, H), lambda i: (i, 0)),
        out_shape=jax.ShapeDtypeStruct((N, H), jnp.float32),
    )(x, WinT, b_in.reshape(1, H))


def _k1b_body(h_ref, wc0T_ref, d0_ref, d1_ref, xs_ref, dinv_ref):
    dinv = lax.rsqrt(1.0 + d0_ref[...] + d1_ref[...])
    dinv_ref[...] = dinv
    xs_ref[...] = dinv * jnp.dot(h_ref[...], wc0T_ref[...],
                                 preferred_element_type=jnp.float32)


def _k1b(h, Wc0T, d0, d1):
    grid = N // ROW_BLK
    return pl.pallas_call(
        _k1b_body,
        grid=(grid,),
        in_specs=[
            pl.BlockSpec((ROW_BLK, H), lambda i: (i, 0)),
            pl.BlockSpec((H, H), lambda i: (0, 0)),
            pl.BlockSpec((ROW_BLK, 1), lambda i: (i, 0)),
            pl.BlockSpec((ROW_BLK, 1), lambda i: (i, 0)),
        ],
        out_specs=[
            pl.BlockSpec((ROW_BLK, H), lambda i: (i, 0)),
            pl.BlockSpec((ROW_BLK, 1), lambda i: (i, 0)),
        ],
        out_shape=[
            jax.ShapeDtypeStruct((N, H), jnp.float32),
            jax.ShapeDtypeStruct((N, 1), jnp.float32),
        ],
    )(h, Wc0T, d0, d1)


def _k3_body(s0_ref, s1_ref, xs_ref, h_ref, b_ref, wT_ref, dinv_ref,
             hn_ref, xsn_ref):
    dinv = dinv_ref[...]
    t = dinv * (s0_ref[...] + s1_ref[...] + xs_ref[...]) + b_ref[...] + h_ref[...]
    hn = jnp.maximum(t, 0.0)
    hn_ref[...] = hn
    xsn_ref[...] = dinv * jnp.dot(hn, wT_ref[...],
                                  preferred_element_type=jnp.float32)


def _k3(S0, S1, xs, h, b, WT, dinv):
    grid = N // ROW_BLK
    return pl.pallas_call(
        _k3_body,
        grid=(grid,),
        in_specs=[
            pl.BlockSpec((ROW_BLK, H), lambda i: (i, 0)),
            pl.BlockSpec((ROW_BLK, H), lambda i: (i, 0)),
            pl.BlockSpec((ROW_BLK, H), lambda i: (i, 0)),
            pl.BlockSpec((ROW_BLK, H), lambda i: (i, 0)),
            pl.BlockSpec((1, H), lambda i: (0, 0)),
            pl.BlockSpec((H, H), lambda i: (0, 0)),
            pl.BlockSpec((ROW_BLK, 1), lambda i: (i, 0)),
        ],
        out_specs=[
            pl.BlockSpec((ROW_BLK, H), lambda i: (i, 0)),
            pl.BlockSpec((ROW_BLK, H), lambda i: (i, 0)),
        ],
        out_shape=[
            jax.ShapeDtypeStruct((N, H), jnp.float32),
            jax.ShapeDtypeStruct((N, H), jnp.float32),
        ],
    )(S0, S1, xs, h, b.reshape(1, H), WT, dinv)


def _k4_body(s0_ref, s1_ref, xs_ref, h_ref, b_ref, dinv_ref, woT_ref,
             bo_ref, o_ref):
    dinv = dinv_ref[...]
    t = dinv * (s0_ref[...] + s1_ref[...] + xs_ref[...]) + b_ref[...] + h_ref[...]
    hn = jnp.maximum(t, 0.0)
    o_ref[...] = jnp.dot(hn, woT_ref[...],
                         preferred_element_type=jnp.float32) + bo_ref[...]


def _k4(S0, S1, xs, h, b, dinv, WoutT, bout):
    grid = N // ROW_BLK
    return pl.pallas_call(
        _k4_body,
        grid=(grid,),
        in_specs=[
            pl.BlockSpec((ROW_BLK, H), lambda i: (i, 0)),
            pl.BlockSpec((ROW_BLK, H), lambda i: (i, 0)),
            pl.BlockSpec((ROW_BLK, H), lambda i: (i, 0)),
            pl.BlockSpec((ROW_BLK, H), lambda i: (i, 0)),
            pl.BlockSpec((1, H), lambda i: (0, 0)),
            pl.BlockSpec((ROW_BLK, 1), lambda i: (i, 0)),
            pl.BlockSpec((H, H), lambda i: (0, 0)),
            pl.BlockSpec((1, H), lambda i: (0, 0)),
        ],
        out_specs=pl.BlockSpec((ROW_BLK, H), lambda i: (i, 0)),
        out_shape=jax.ShapeDtypeStruct((N, H), jnp.float32),
    )(S0, S1, xs, h, b.reshape(1, H), dinv, WoutT, bout.reshape(1, H))


def _edge_mlp_body(ea_ref, w1_ref, b1_ref, w2_ref, b2_ref, o_ref):
    a = jnp.dot(ea_ref[...], w1_ref[...], preferred_element_type=jnp.float32)
    a = jnp.maximum(a + b1_ref[...], 0.0)
    sc = jnp.dot(a, w2_ref[...], preferred_element_type=jnp.float32) + b2_ref[...]
    o_ref[...] = jax.nn.sigmoid(sc)


def _edge_mlp(edge_attr, We1, be1, We2, be2):
    grid = E // EDGE_BLK
    out = pl.pallas_call(
        _edge_mlp_body,
        grid=(grid,),
        in_specs=[
            pl.BlockSpec((EDGE_BLK, 16), lambda i: (i, 0)),
            pl.BlockSpec((16, 96), lambda i: (0, 0)),
            pl.BlockSpec((1, 96), lambda i: (0, 0)),
            pl.BlockSpec((96, 1), lambda i: (0, 0)),
            pl.BlockSpec((1, 1), lambda i: (0, 0)),
        ],
        out_specs=pl.BlockSpec((EDGE_BLK, 1), lambda i: (i, 0)),
        out_shape=jax.ShapeDtypeStruct((E, 1), jnp.float32),
    )(edge_attr, We1.T, be1.reshape(1, 96), We2.T, be2.reshape(1, 1))
    return out[:, 0]


def kernel(x, edge_index, edge_attr, Win, b_in, We1, be1, We2, be2,
           Wc0, bc0, Wc1, bc1, Wc2, bc2, Wout, bout):
    src = edge_index[0].astype(jnp.int32)
    dst = edge_index[1].astype(jnp.int32)

    ew = _edge_mlp(edge_attr, We1, be1, We2, be2)

    pad = EPAD - E
    zi = jnp.arange(pad, dtype=jnp.int32) % N
    srcr = jnp.concatenate([src, zi]).reshape(NW, NCHUNK, C)
    dstr = jnp.concatenate([dst, zi]).reshape(NW, NCHUNK, C)
    ewr = jnp.concatenate([ew, jnp.zeros((pad,), jnp.float32)]).reshape(
        NW, NCHUNK, C)

    h = _k1a(x, Win.T, b_in)

    degp = _deg_kernel(dstr, ewr)
    d0 = degp[0, :N].reshape(N, 1)
    d1 = degp[1, :N].reshape(N, 1)

    xs, dinv = _k1b(h, Wc0.T, d0, d1)

    def pack_u16(idx_flat):
        a = idx_flat.reshape(-1, 2, 16)
        words = a[:, 0, :] | (a[:, 1, :] << 16)
        return words.reshape(NW, EW // 256, 128)

    srcp = pack_u16(jnp.concatenate([src, zi]))
    dstp = pack_u16(jnp.concatenate([dst, zi]))

    for (W_next, b_cur) in ((Wc1, bc0), (Wc2, bc1)):
        S = _spmm_kernel(xs, srcp, dstp, ewr)
        h, xs = _k3(S[0], S[1], xs, h, b_cur, W_next.T, dinv)

    S = _spmm_kernel(xs, srcp, dstp, ewr)
    return _k4(S[0], S[1], xs, h, bc2, dinv, Wout.T, bout)


# speedup vs baseline: 1.1051x; 1.0919x over previous
import functools
import jax
import jax.numpy as jnp
from jax import lax
from jax.experimental import pallas as pl
from jax.experimental.pallas import tpu as pltpu
from jax.experimental.pallas import tpu_sc as plsc

N, E, H = 10000, 320000, 128
NC, NS, NW = 2, 16, 32
C = 128
EW = 10240
NCHUNK = EW // C
C2 = 64
NCH2 = EW // C2
EPAD = NW * EW
NPAD = NS * 640
DEGW = NPAD

ROW_BLK = 2000
EDGE_BLK = 2000

_sc_mesh = plsc.VectorSubcoreMesh(core_axis_name="c", subcore_axis_name="s")


@functools.partial(
    pl.kernel,
    out_type=jax.ShapeDtypeStruct((NC, DEGW), jnp.float32),
    mesh=_sc_mesh,
    scratch_types=[
        pltpu.VMEM((NCHUNK, C), jnp.int32),
        pltpu.VMEM((NCHUNK, C), jnp.float32),
        pltpu.VMEM_SHARED((DEGW,), jnp.float32),
        pltpu.VMEM((640,), jnp.float32),
    ],
)
def _deg_kernel(dst_hbm, ew_hbm, out_hbm, dstv, ewv, sdeg, zbuf):
    c = lax.axis_index("c")
    s = lax.axis_index("s")
    w = s * NC + c
    pltpu.sync_copy(dst_hbm.at[w], dstv)
    pltpu.sync_copy(ew_hbm.at[w], ewv)

    zvec = jnp.zeros((16,), jnp.float32)

    def zstore(i, carry):
        zbuf[pl.ds(i * 16, 16)] = zvec
        return carry

    lax.fori_loop(0, 40, zstore, 0)
    pltpu.sync_copy(zbuf, sdeg.at[pl.ds(s * 640, 640)])
    plsc.subcore_barrier()

    def chunk(j, carry):
        pltpu.sync_copy(ewv.at[j], sdeg.at[dstv.at[j]], add=True)
        return carry

    lax.fori_loop(0, NCHUNK, chunk, 0)
    plsc.subcore_barrier()
    pltpu.sync_copy(sdeg.at[pl.ds(s * 640, 640)], out_hbm.at[c, pl.ds(s * 640, 640)])


@functools.partial(
    pl.kernel,
    out_type=jax.ShapeDtypeStruct((NC, NPAD, H), jnp.float32),
    mesh=_sc_mesh,
    scratch_types=[
        pltpu.VMEM((EW // 256, 128), jnp.int32),
        pltpu.VMEM((EW // 256, 128), jnp.int32),
        pltpu.VMEM((NCHUNK, C), jnp.float32),
        pltpu.VMEM((C2, H), jnp.float32),
        pltpu.VMEM((C2, H), jnp.float32),
        pltpu.VMEM((C2, H), jnp.float32),
        pltpu.VMEM((C2,), jnp.int32),
        pltpu.VMEM((C2,), jnp.int32),
        pltpu.VMEM((C2,), jnp.int32),
        pltpu.VMEM((C2,), jnp.int32),
        pltpu.VMEM((C2,), jnp.int32),
        pltpu.VMEM((C2,), jnp.int32),
        pltpu.VMEM_SHARED((NPAD, H), jnp.float32),
        pltpu.SemaphoreType.DMA,
        pltpu.SemaphoreType.DMA,
        pltpu.SemaphoreType.DMA,
        pltpu.SemaphoreType.DMA,
        pltpu.SemaphoreType.DMA,
        pltpu.SemaphoreType.DMA,
    ],
)
def _spmm_kernel(xs_hbm, srcp_hbm, dstp_hbm, ewr_hbm, out_hbm,
                 srcp, dstp, ewv, r0, r1, r2, st0, st1, st2,
                 dt0, dt1, dt2, sacc,
                 sg0, sg1, sg2, ss0, ss1, ss2):
    c = lax.axis_index("c")
    s = lax.axis_index("s")
    w = s * NC + c
    pltpu.async_copy(srcp_hbm.at[w], srcp, sg0)
    pltpu.async_copy(dstp_hbm.at[w], dstp, sg0)
    pltpu.async_copy(ewr_hbm.at[w], ewv, sg0)

    rowss = (r0, r1, r2)
    stages = (st0, st1, st2)
    dstages = (dt0, dt1, dt2)
    sgs = (sg0, sg1, sg2)
    sss = (ss0, ss1, ss2)
    zvec = jnp.zeros((16,), jnp.float32)
    m16 = jnp.full((16,), 0xFFFF, jnp.int32)
    sh16 = jnp.full((16,), 16, jnp.int32)

    def zrow(i, carry):
        for k in range(8):
            r0[i, pl.ds(k * 16, 16)] = zvec
        return carry

    lax.fori_loop(0, C2, zrow, 0)
    for t in range(10):
        pltpu.async_copy(r0, sacc.at[pl.ds(s * 640 + t * C2, C2)], sg1)
    for t in range(10):
        pltpu.make_async_copy(r0, sacc.at[pl.ds(s * 640, C2)], sg1).wait()
    pltpu.make_async_copy(srcp_hbm.at[0], srcp, sg0).wait()
    pltpu.make_async_copy(dstp_hbm.at[0], dstp, sg0).wait()
    pltpu.make_async_copy(ewr_hbm.at[0], ewv, sg0).wait()
    plsc.subcore_barrier()

    gdn = lax.GatherDimensionNumbers(
        offset_dims=(), collapsed_slice_dims=(0,), start_index_map=(0,))

    def unpack_idx(pk, j, blk):
        word0 = j * 32 + blk * 16
        v = pk[word0 // 128, pl.ds(word0 % 128, 16)]
        lo = jnp.bitwise_and(v, m16)
        hi = lax.shift_right_logical(v, sh16)
        return lo, hi

    def start_gather(j, r):
        sb = stages[r]
        for blk in range(2):
            lo, hi = unpack_idx(srcp, j, blk)
            sb[pl.ds(blk * 32, 16)] = lo
            sb[pl.ds(blk * 32 + 16, 16)] = hi
        pltpu.async_copy(xs_hbm.at[sb], rowss[r], sgs[r])

    def wait_gather(r):
        pltpu.make_async_copy(xs_hbm.at[stages[r]], rowss[r], sgs[r]).wait()

    def wait_scatter(r):
        pltpu.make_async_copy(rowss[r], sacc.at[stages[r]], sss[r]).wait()

    def scale(j, r):
        rb = rowss[r]
        for g in range(C2 // 16):
            ew16 = ewv[j // 2, pl.ds((j % 2) * C2 + g * 16, 16)]
            for l in range(16):
                lidx = jnp.full((16, 1), l, jnp.int32)
                ewvec = lax.gather(
                    ew16, lidx, gdn, (1,),
                    mode=lax.GatherScatterMode.PROMISE_IN_BOUNDS)
                e = g * 16 + l
                for k in range(8):
                    sl = pl.ds(k * 16, 16)
                    rb[e, sl] = rb[e, sl] * ewvec

    def start_scatter(j, r):
        db = dstages[r]
        for blk in range(2):
            lo, hi = unpack_idx(dstp, j, blk)
            db[pl.ds(blk * 32, 16)] = lo
            db[pl.ds(blk * 32 + 16, 16)] = hi
        pltpu.async_copy(rowss[r], sacc.at[db], sss[r], add=True)

    def body(i, r, first=False, last=False):
        r2_ = (r + 2) % 3
        wait_gather(r)
        scale(i, r)
        start_scatter(i, r)
        if not last:
            if not first:
                wait_scatter(r2_)
            start_gather(i + 2, r2_)

    start_gather(0, 0)
    start_gather(1, 1)
    body(0, 0, first=True)
    body(1, 1)

    def steady(t, carry):
        i0 = 2 + 3 * t
        for b in range(3):
            body(i0 + b, (2 + b) % 3)
        return carry

    lax.fori_loop(0, (NCH2 - 4) // 3, steady, 0)

    body(NCH2 - 2, (NCH2 - 2) % 3, last=True)
    body(NCH2 - 1, (NCH2 - 1) % 3, last=True)
    for r in range(3):
        wait_scatter(r)

    plsc.subcore_barrier()
    for t in range(5):
        sl = pl.ds(s * 640 + t * 128, 128)
        pltpu.sync_copy(sacc.at[sl], out_hbm.at[c, sl])


def _k1a_body(x_ref, winT_ref, bin_ref, h0_ref):
    h = jnp.dot(x_ref[...], winT_ref[...], preferred_element_type=jnp.float32)
    h0_ref[...] = jnp.maximum(h + bin_ref[...], 0.0)


def _k1a(x, WinT, b_in):
    grid = N // ROW_BLK
    return pl.pallas_call(
        _k1a_body,
        grid=(grid,),
        in_specs=[
            pl.BlockSpec((ROW_BLK, H), lambda i: (i, 0)),
            pl.BlockSpec((H, H), lambda i: (0, 0)),
            pl.BlockSpec((1, H), lambda i: (0, 0)),
        ],
        out_specs=pl.BlockSpec((ROW_BLK, H), lambda i: (i, 0)),
        out_shape=jax.ShapeDtypeStruct((N, H), jnp.float32),
    )(x, WinT, b_in.reshape(1, H))


def _k1b_body(h_ref, wc0T_ref, d0_ref, d1_ref, xs_ref, dinv_ref):
    dinv = lax.rsqrt(1.0 + d0_ref[...] + d1_ref[...])
    dinv_ref[...] = dinv
    xs_ref[...] = dinv * jnp.dot(h_ref[...], wc0T_ref[...],
                                 preferred_element_type=jnp.float32)


def _k1b(h, Wc0T, d0, d1):
    grid = N // ROW_BLK
    return pl.pallas_call(
        _k1b_body,
        grid=(grid,),
        in_specs=[
            pl.BlockSpec((ROW_BLK, H), lambda i: (i, 0)),
            pl.BlockSpec((H, H), lambda i: (0, 0)),
            pl.BlockSpec((ROW_BLK, 1), lambda i: (i, 0)),
            pl.BlockSpec((ROW_BLK, 1), lambda i: (i, 0)),
        ],
        out_specs=[
            pl.BlockSpec((ROW_BLK, H), lambda i: (i, 0)),
            pl.BlockSpec((ROW_BLK, 1), lambda i: (i, 0)),
        ],
        out_shape=[
            jax.ShapeDtypeStruct((N, H), jnp.float32),
            jax.ShapeDtypeStruct((N, 1), jnp.float32),
        ],
    )(h, Wc0T, d0, d1)


def _k3_body(s0_ref, s1_ref, xs_ref, h_ref, b_ref, wT_ref, dinv_ref,
             hn_ref, xsn_ref):
    dinv = dinv_ref[...]
    t = dinv * (s0_ref[...] + s1_ref[...] + xs_ref[...]) + b_ref[...] + h_ref[...]
    hn = jnp.maximum(t, 0.0)
    hn_ref[...] = hn
    xsn_ref[...] = dinv * jnp.dot(hn, wT_ref[...],
                                  preferred_element_type=jnp.float32)


def _k3(S0, S1, xs, h, b, WT, dinv):
    grid = N // ROW_BLK
    return pl.pallas_call(
        _k3_body,
        grid=(grid,),
        in_specs=[
            pl.BlockSpec((ROW_BLK, H), lambda i: (i, 0)),
            pl.BlockSpec((ROW_BLK, H), lambda i: (i, 0)),
            pl.BlockSpec((ROW_BLK, H), lambda i: (i, 0)),
            pl.BlockSpec((ROW_BLK, H), lambda i: (i, 0)),
            pl.BlockSpec((1, H), lambda i: (0, 0)),
            pl.BlockSpec((H, H), lambda i: (0, 0)),
            pl.BlockSpec((ROW_BLK, 1), lambda i: (i, 0)),
        ],
        out_specs=[
            pl.BlockSpec((ROW_BLK, H), lambda i: (i, 0)),
            pl.BlockSpec((ROW_BLK, H), lambda i: (i, 0)),
        ],
        out_shape=[
            jax.ShapeDtypeStruct((N, H), jnp.float32),
            jax.ShapeDtypeStruct((N, H), jnp.float32),
        ],
    )(S0, S1, xs, h, b.reshape(1, H), WT, dinv)


def _k4_body(s0_ref, s1_ref, xs_ref, h_ref, b_ref, dinv_ref, woT_ref,
             bo_ref, o_ref):
    dinv = dinv_ref[...]
    t = dinv * (s0_ref[...] + s1_ref[...] + xs_ref[...]) + b_ref[...] + h_ref[...]
    hn = jnp.maximum(t, 0.0)
    o_ref[...] = jnp.dot(hn, woT_ref[...],
                         preferred_element_type=jnp.float32) + bo_ref[...]


def _k4(S0, S1, xs, h, b, dinv, WoutT, bout):
    grid = N // ROW_BLK
    return pl.pallas_call(
        _k4_body,
        grid=(grid,),
        in_specs=[
            pl.BlockSpec((ROW_BLK, H), lambda i: (i, 0)),
            pl.BlockSpec((ROW_BLK, H), lambda i: (i, 0)),
            pl.BlockSpec((ROW_BLK, H), lambda i: (i, 0)),
            pl.BlockSpec((ROW_BLK, H), lambda i: (i, 0)),
            pl.BlockSpec((1, H), lambda i: (0, 0)),
            pl.BlockSpec((ROW_BLK, 1), lambda i: (i, 0)),
            pl.BlockSpec((H, H), lambda i: (0, 0)),
            pl.BlockSpec((1, H), lambda i: (0, 0)),
        ],
        out_specs=pl.BlockSpec((ROW_BLK, H), lambda i: (i, 0)),
        out_shape=jax.ShapeDtypeStruct((N, H), jnp.float32),
    )(S0, S1, xs, h, b.reshape(1, H), dinv, WoutT, bout.reshape(1, H))


def _edge_mlp_body(ea_ref, w1_ref, b1_ref, w2_ref, b2_ref, o_ref):
    a = jnp.dot(ea_ref[...], w1_ref[...], preferred_element_type=jnp.float32)
    a = jnp.maximum(a + b1_ref[...], 0.0)
    sc = jnp.dot(a, w2_ref[...], preferred_element_type=jnp.float32) + b2_ref[...]
    o_ref[...] = jax.nn.sigmoid(sc)


def _edge_mlp(edge_attr, We1, be1, We2, be2):
    eap = edge_attr.reshape(E // 8, 128)
    w1bd = jnp.kron(jnp.eye(8, dtype=jnp.float32), We1.T)
    b1bd = jnp.tile(be1, 8).reshape(1, 768)
    w2bd = jnp.kron(jnp.eye(8, dtype=jnp.float32), We2.T)
    b2bd = jnp.tile(be2, 8).reshape(1, 8)
    grid = (E // 8) // EDGE_BLK
    out = pl.pallas_call(
        _edge_mlp_body,
        grid=(grid,),
        in_specs=[
            pl.BlockSpec((EDGE_BLK, 128), lambda i: (i, 0)),
            pl.BlockSpec((128, 768), lambda i: (0, 0)),
            pl.BlockSpec((1, 768), lambda i: (0, 0)),
            pl.BlockSpec((768, 8), lambda i: (0, 0)),
            pl.BlockSpec((1, 8), lambda i: (0, 0)),
        ],
        out_specs=pl.BlockSpec((EDGE_BLK, 8), lambda i: (i, 0)),
        out_shape=jax.ShapeDtypeStruct((E // 8, 8), jnp.float32),
    )(eap, w1bd, b1bd, w2bd, b2bd)
    return out.reshape(E)


def kernel(x, edge_index, edge_attr, Win, b_in, We1, be1, We2, be2,
           Wc0, bc0, Wc1, bc1, Wc2, bc2, Wout, bout):
    src = edge_index[0].astype(jnp.int32)
    dst = edge_index[1].astype(jnp.int32)

    ew = _edge_mlp(edge_attr, We1, be1, We2, be2)

    pad = EPAD - E
    zi = jnp.arange(pad, dtype=jnp.int32) % N
    srcr = jnp.concatenate([src, zi]).reshape(NW, NCHUNK, C)
    dstr = jnp.concatenate([dst, zi]).reshape(NW, NCHUNK, C)
    ewr = jnp.concatenate([ew, jnp.zeros((pad,), jnp.float32)]).reshape(
        NW, NCHUNK, C)

    h = _k1a(x, Win.T, b_in)

    degp = _deg_kernel(dstr, ewr)
    d0 = degp[0, :N].reshape(N, 1)
    d1 = degp[1, :N].reshape(N, 1)

    xs, dinv = _k1b(h, Wc0.T, d0, d1)

    def pack_u16(idx_flat):
        a = idx_flat.reshape(-1, 2, 16)
        words = a[:, 0, :] | (a[:, 1, :] << 16)
        return words.reshape(NW, EW // 256, 128)

    srcp = pack_u16(jnp.concatenate([src, zi]))
    dstp = pack_u16(jnp.concatenate([dst, zi]))

    for (W_next, b_cur) in ((Wc1, bc0), (Wc2, bc1)):
        S = _spmm_kernel(xs, srcp, dstp, ewr)
        h, xs = _k3(S[0], S[1], xs, h, b_cur, W_next.T, dinv)

    S = _spmm_kernel(xs, srcp, dstp, ewr)
    return _k4(S[0], S[1], xs, h, bc2, dinv, Wout.T, bout)
